# Initial kernel scaffold; baseline (speedup 1.0000x reference)
import jax
import jax.numpy as jnp
from jax.experimental import pallas as pl


def kernel(x, edge_index, edge_attr, Wg1, mu1, sigma1, Wroot1, b1, Wg2, mu2, sigma2, Wroot2, b2, Wfc, bfc):
    raise NotImplementedError("write your pallas kernel here")


# speedup vs baseline: 2.3411x; 2.3411x over previous
import jax
import jax.numpy as jnp
from jax import lax
from jax.experimental import pallas as pl
from jax.experimental.pallas import tpu as pltpu
from jax.experimental.pallas import tpu_sc as plsc

_EPS = 1e-15

_N = 10000
_E = 160000
_H = 128
_K = 3
_GW = _K * _H

_EPT = _E // 32
_C = 40
_NCH = _EPT // _C
_WSTR = _EPT + 8


def _proj_body(x_ref, wg_ref, wr_ref, xg_ref, xr_ref):
    x = x_ref[...]
    xg_ref[...] = jnp.dot(x, wg_ref[...], preferred_element_type=jnp.float32)
    xr_ref[...] = jnp.dot(x, wr_ref[...], preferred_element_type=jnp.float32)


def _proj(x, wg, wr):
    n, f = x.shape
    bn = 1000
    return pl.pallas_call(
        _proj_body,
        grid=(n // bn,),
        in_specs=[
            pl.BlockSpec((bn, f), lambda i: (i, 0)),
            pl.BlockSpec((f, _GW), lambda i: (0, 0)),
            pl.BlockSpec((f, _H), lambda i: (0, 0)),
        ],
        out_specs=[
            pl.BlockSpec((bn, _GW), lambda i: (i, 0)),
            pl.BlockSpec((bn, _H), lambda i: (i, 0)),
        ],
        out_shape=[
            jax.ShapeDtypeStruct((n, _GW), jnp.float32),
            jax.ShapeDtypeStruct((n, _H), jnp.float32),
        ],
    )(x, wg, wr)


def _weights_body(a0_ref, a1_ref, mu_ref, sg_ref, w_ref):
    a0 = a0_ref[...]
    a1 = a1_ref[...]
    for k in range(_K):
        s0 = sg_ref[k, 0]
        s1 = sg_ref[k, 1]
        c0 = -0.5 / (_EPS + s0 * s0)
        c1 = -0.5 / (_EPS + s1 * s1)
        d0 = a0 - mu_ref[k, 0]
        d1 = a1 - mu_ref[k, 1]
        w_ref[k] = jnp.exp(d0 * d0 * c0 + d1 * d1 * c1)


def _edge_weights(edge_attr, mu, sigma):
    r = _E // 128
    a0 = edge_attr[:, 0].reshape(r, 128)
    a1 = edge_attr[:, 1].reshape(r, 128)
    w = pl.pallas_call(
        _weights_body,
        grid=(1,),
        in_specs=[
            pl.BlockSpec((r, 128), lambda i: (0, 0)),
            pl.BlockSpec((r, 128), lambda i: (0, 0)),
            pl.BlockSpec(memory_space=pltpu.SMEM),
            pl.BlockSpec(memory_space=pltpu.SMEM),
        ],
        out_specs=pl.BlockSpec((_K, r, 128), lambda i: (0, 0, 0)),
        out_shape=jax.ShapeDtypeStruct((_K, r, 128), jnp.float32),
    )(a0, a1, mu, sigma)
    return w.reshape(_K * _E)


def _combine_proj_body(a_ref, cnt_ref, xr_ref, b_ref, wg_ref, wr_ref,
                       h_ref, xg_ref, xr2_ref):
    h = jnp.maximum(
        a_ref[...] / jnp.maximum(cnt_ref[...], 1.0) + xr_ref[...]
        + b_ref[...], 0.0)
    h_ref[...] = h
    xg_ref[...] = jnp.dot(h, wg_ref[...], preferred_element_type=jnp.float32)
    xr2_ref[...] = jnp.dot(h, wr_ref[...], preferred_element_type=jnp.float32)


def _combine_proj(agg, cnt, xr, b, wg, wr):
    bn = 1000
    return pl.pallas_call(
        _combine_proj_body,
        grid=(_N // bn,),
        in_specs=[
            pl.BlockSpec((bn, _H), lambda i: (i, 0)),
            pl.BlockSpec((bn, 1), lambda i: (i, 0)),
            pl.BlockSpec((bn, _H), lambda i: (i, 0)),
            pl.BlockSpec((1, _H), lambda i: (0, 0)),
            pl.BlockSpec((_H, _GW), lambda i: (0, 0)),
            pl.BlockSpec((_H, _H), lambda i: (0, 0)),
        ],
        out_specs=[
            pl.BlockSpec((bn, _H), lambda i: (i, 0)),
            pl.BlockSpec((bn, _GW), lambda i: (i, 0)),
            pl.BlockSpec((bn, _H), lambda i: (i, 0)),
        ],
        out_shape=[
            jax.ShapeDtypeStruct((_N, _H), jnp.float32),
            jax.ShapeDtypeStruct((_N, _GW), jnp.float32),
            jax.ShapeDtypeStruct((_N, _H), jnp.float32),
        ],
    )(agg, cnt, xr, b.reshape(1, _H), wg, wr)


def _fc_body(h_ref, wfc_ref, bfc_ref, o_ref):
    o = jnp.maximum(
        jnp.dot(h_ref[...], wfc_ref[...], preferred_element_type=jnp.float32)
        + bfc_ref[...], 0.0)
    m = jnp.max(o, axis=1, keepdims=True)
    lse = m + jnp.log(jnp.sum(jnp.exp(o - m), axis=1, keepdims=True))
    o_ref[...] = o - lse


def _fc(h, wfc, bfc):
    bn = 1000
    return pl.pallas_call(
        _fc_body,
        grid=(_N // bn,),
        in_specs=[
            pl.BlockSpec((bn, _H), lambda i: (i, 0)),
            pl.BlockSpec((_H, 2), lambda i: (0, 0)),
            pl.BlockSpec((1, 2), lambda i: (0, 0)),
        ],
        out_specs=pl.BlockSpec((bn, 2), lambda i: (i, 0)),
        out_shape=jax.ShapeDtypeStruct((_N, 2), jnp.float32),
    )(h, wfc, bfc.reshape(1, 2))


def _sc_body(xg_hbm, src_hbm, w_hbm, out_hbm,
             src_all, w_all, g0, g1, msg, sj0, sj1, sem0, sem1):
    c = lax.axis_index("c")
    s = lax.axis_index("s")
    t = c * 16 + s

    pltpu.sync_copy(src_hbm.at[t], src_all)
    for kk in range(_K):
        pltpu.sync_copy(w_hbm.at[pl.ds(kk * _E + t * _EPT, _EPT)],
                        w_all.at[pl.ds(kk * _WSTR, _EPT)])

    def _rows8(j, gbuf, gbase):
        eb = j * _C + gbase
        wv0 = w_all[pl.ds(eb, 16)]
        wv1 = w_all[pl.ds(_WSTR + eb, 16)]
        wv2 = w_all[pl.ds(2 * _WSTR + eb, 16)]
        for ii in range(8):
            i = gbase + ii
            w0 = wv0[ii]
            w1 = wv1[ii]
            w2 = wv2[ii]
            for cc in range(_H // 16):
                acc = (w0 * gbuf[i, pl.ds(cc * 16, 16)]
                       + w1 * gbuf[i, pl.ds(_H + cc * 16, 16)]
                       + w2 * gbuf[i, pl.ds(2 * _H + cc * 16, 16)])
                msg[i, pl.ds(cc * 16, 16)] = acc

    def process_chunk(j, gbuf):
        def grp(g, _):
            _rows8(j, gbuf, g * 8)
            return 0
        lax.fori_loop(0, _C // 8, grp, 0)
        pltpu.sync_copy(msg, out_hbm.at[pl.ds(t * _EPT + j * _C, _C)])

    def _fill_src(buf, j):
        buf[pl.ds(0, 16)] = src_all[j, pl.ds(0, 16)]
        buf[pl.ds(16, 16)] = src_all[j, pl.ds(16, 16)]
        buf[pl.ds(24, 16)] = src_all[j, pl.ds(24, 16)]

    def start(j, gbuf, sem, sj):
        _fill_src(sj, j)
        pltpu.async_copy(xg_hbm.at[sj], gbuf, sem)

    def wait(gbuf, sem, sj):
        pltpu.make_async_copy(xg_hbm.at[sj], gbuf, sem).wait()

    start(0, g0, sem0, sj0)
    start(1, g1, sem1, sj1)

    def pair(tt, _):
        j0 = 2 * tt
        wait(g0, sem0, sj0)
        process_chunk(j0, g0)

        @pl.when(j0 + 2 < _NCH)
        def _():
            start(j0 + 2, g0, sem0, sj0)
        wait(g1, sem1, sj1)
        process_chunk(j0 + 1, g1)

        @pl.when(j0 + 3 < _NCH)
        def _():
            start(j0 + 3, g1, sem1, sj1)
        return 0
    lax.fori_loop(0, _NCH // 2, pair, 0)
    wait(g0, sem0, sj0)
    process_chunk(_NCH - 1, g0)


_SC_MSG = pl.kernel(
    _sc_body,
    out_type=jax.ShapeDtypeStruct((_E, _H), jnp.float32),
    mesh=plsc.VectorSubcoreMesh(core_axis_name="c", subcore_axis_name="s"),
    scratch_types=[
        pltpu.VMEM((_NCH, _C), jnp.int32),
        pltpu.VMEM((_K * _WSTR,), jnp.float32),
        pltpu.VMEM((_C, _GW), jnp.float32),
        pltpu.VMEM((_C, _GW), jnp.float32),
        pltpu.VMEM((_C, _H), jnp.float32),
        pltpu.VMEM((_C,), jnp.int32),
        pltpu.VMEM((_C,), jnp.int32),
        pltpu.SemaphoreType.DMA,
        pltpu.SemaphoreType.DMA,
    ],
    compiler_params=pltpu.CompilerParams(use_tc_tiling_on_sc=False),
)


def kernel(x, edge_index, edge_attr, Wg1, mu1, sigma1, Wroot1, b1,
           Wg2, mu2, sigma2, Wroot2, b2, Wfc, bfc):
    src_r = edge_index[0].reshape(32, _NCH, _C)
    dst = edge_index[1]

    w1 = _edge_weights(edge_attr, mu1, sigma1)
    w2 = _edge_weights(edge_attr, mu2, sigma2)

    cnt = jax.ops.segment_sum(jnp.ones((_E,), jnp.float32), dst,
                              num_segments=_N).reshape(_N, 1)

    xg1, xr1 = _proj(x, Wg1, Wroot1)

    msg1 = _SC_MSG(xg1, src_r, w1)
    agg1 = jax.ops.segment_sum(msg1, dst, num_segments=_N)
    _h1, xg2, xr2 = _combine_proj(agg1, cnt, xr1, b1, Wg2, Wroot2)

    msg2 = _SC_MSG(xg2, src_r, w2)
    agg2 = jax.ops.segment_sum(msg2, dst, num_segments=_N)
    h2, _xg3, _xr3 = _combine_proj(agg2, cnt, xr2, b2,
                                   jnp.zeros((_H, _GW), jnp.float32),
                                   jnp.zeros((_H, _H), jnp.float32))
    return _fc(h2, Wfc, bfc)
